# Initial kernel scaffold; baseline (speedup 1.0000x reference)
import jax
import jax.numpy as jnp
from jax.experimental import pallas as pl


def kernel(region_id, x, y, arrival_time, departure_time, region_table, Ws, bs, w0, b0, Wt, Bt):
    raise NotImplementedError("write your pallas kernel here")


# speedup vs baseline: 2.3373x; 2.3373x over previous
import functools

import jax
import jax.numpy as jnp
import ml_dtypes
import numpy as np
from jax import lax
from jax.experimental import pallas as pl
from jax.experimental.pallas import tpu as pltpu
from jax.experimental.pallas import tpu_sc as plsc

_D = 128
_NUM_SCALES = _D // 6
_LAMBDA_MIN = 0.01
_LAMBDA_MAX = 10.0
_CH = 128
_G = 5
_T = 2048


def _sc_gather(table, ids_flat):
    info = plsc.get_sparse_core_info()
    nc, ns = info.num_cores, info.num_subcores
    nw = nc * ns
    n = ids_flat.shape[0]
    per_w = n // nw
    k = per_w // _CH
    steps = k // _G
    mesh = plsc.VectorSubcoreMesh(core_axis_name="c", subcore_axis_name="s")

    @functools.partial(
        pl.kernel, mesh=mesh,
        out_type=jax.ShapeDtypeStruct((n, _D), jnp.float32),
        scratch_types=[
            pltpu.VMEM((_G * _CH,), jnp.int32),
            pltpu.VMEM((_G, _CH, _D), jnp.float32),
            pltpu.SemaphoreType.DMA,
        ],
    )
    def gk(ids_hbm, table_hbm, out_hbm, idx_v, rows_v, sem):
        wid = lax.axis_index("s") * nc + lax.axis_index("c")
        tbase = wid * per_w

        def body(i, carry):
            pltpu.sync_copy(
                ids_hbm.at[pl.ds(tbase + i * (_G * _CH), _G * _CH)], idx_v)
            cps = [
                pltpu.async_copy(
                    table_hbm.at[idx_v.at[pl.ds(b * _CH, _CH)]],
                    rows_v.at[b], sem)
                for b in range(_G)
            ]
            for b in range(_G):
                cps[b].wait()
                pltpu.sync_copy(
                    rows_v.at[b],
                    out_hbm.at[pl.ds(tbase + (i * _G + b) * _CH, _CH)])
            return carry

        lax.fori_loop(0, steps, body, 0)

    return gk(ids_flat, table)


def _tc_body(x_ref, y_ref, a_ref, d_ref, k_ref, wsin_ref, wcos_ref, bs_ref,
             wall_ref, ball_ref, reg_ref, out_ref):
    xv = x_ref[...].astype(jnp.bfloat16).astype(jnp.float32)
    yv = y_ref[...].astype(jnp.bfloat16).astype(jnp.float32)
    z = xv * k_ref[0:1, :] + yv * k_ref[1:2, :]
    nrot = jnp.floor(z * 0.15915494309189535 + 0.5)
    z = (z - nrot * 6.28125) - nrot * 1.9353071795864769e-3
    loc = jnp.dot(jnp.sin(z).astype(jnp.bfloat16), wsin_ref[...],
                  preferred_element_type=jnp.float32)
    loc += jnp.dot(jnp.cos(z).astype(jnp.bfloat16), wcos_ref[...],
                   preferred_element_type=jnp.float32)
    out_ref[:, 0:_D] = loc + bs_ref[...]
    col = lax.broadcasted_iota(jnp.int32, (1, _D), 1)
    aa = a_ref[...] * wall_ref[...] + ball_ref[...]
    out_ref[:, _D:2 * _D] = jnp.where(col == 0, aa, jnp.sin(aa))
    dd = d_ref[...] * wall_ref[...] + ball_ref[...]
    out_ref[:, 2 * _D:3 * _D] = jnp.where(col == 0, dd, jnp.sin(dd))
    out_ref[:, 3 * _D:4 * _D] = reg_ref[...]


def kernel(region_id, x, y, arrival_time, departure_time, region_table,
           Ws, bs, w0, b0, Wt, Bt):
    b_dim, s_dim = x.shape
    n = b_dim * s_dim

    reg = _sc_gather(region_table, region_id.reshape(n))

    scales = _LAMBDA_MIN * (_LAMBDA_MAX / _LAMBDA_MIN) ** (
        np.arange(_NUM_SCALES, dtype=np.float32) / max(_NUM_SCALES - 1, 1))
    angles = np.array([0.0, 2.0 * np.pi / 3.0, 4.0 * np.pi / 3.0],
                      dtype=np.float32)
    dirs = np.stack([np.cos(angles), np.sin(angles)], axis=-1)
    dirs_b = dirs.astype(ml_dtypes.bfloat16).astype(np.float32)
    kmat = (dirs_b[None, :, :].astype(np.float64)
            / scales[:, None, None].astype(np.float64)).reshape(
        3 * _NUM_SCALES, 2).T.astype(np.float32)
    kmat = jnp.asarray(kmat)

    ws3 = Ws.reshape(_NUM_SCALES, 6, _D)
    ws_sin = ws3[:, :3, :].reshape(3 * _NUM_SCALES, _D).astype(jnp.bfloat16)
    ws_cos = ws3[:, 3:, :].reshape(3 * _NUM_SCALES, _D).astype(jnp.bfloat16)
    w_all = jnp.concatenate([w0[None], Wt]).reshape(1, _D)
    b_all = jnp.concatenate([b0[None], Bt]).reshape(1, _D)

    col = lambda v: v.reshape(n, 1)
    grid = n // _T
    tok_spec = pl.BlockSpec((_T, 1), lambda i: (i, 0))
    full = lambda r, c: pl.BlockSpec((r, c), lambda i: (0, 0))

    out = pl.pallas_call(
        _tc_body,
        grid=(grid,),
        in_specs=[
            tok_spec, tok_spec, tok_spec, tok_spec,
            full(2, 3 * _NUM_SCALES),
            full(3 * _NUM_SCALES, _D),
            full(3 * _NUM_SCALES, _D),
            full(1, _D), full(1, _D), full(1, _D),
            pl.BlockSpec((_T, _D), lambda i: (i, 0)),
        ],
        out_specs=pl.BlockSpec((_T, 4 * _D), lambda i: (i, 0)),
        out_shape=jax.ShapeDtypeStruct((n, 4 * _D), jnp.float32),
    )(col(x), col(y), col(arrival_time), col(departure_time),
      kmat, ws_sin, ws_cos, bs.reshape(1, _D), w_all, b_all, reg)

    return out.reshape(b_dim, s_dim, 4 * _D)


# speedup vs baseline: 2.8929x; 1.2377x over previous
import functools

import jax
import jax.numpy as jnp
import ml_dtypes
import numpy as np
from jax import lax
from jax.experimental import pallas as pl
from jax.experimental.pallas import tpu as pltpu
from jax.experimental.pallas import tpu_sc as plsc

_D = 128
_NUM_SCALES = _D // 6
_LAMBDA_MIN = 0.01
_LAMBDA_MAX = 10.0
_CH = 128
_G = 5
_T = 2048


def _sc_gather(table, ids_flat):
    info = plsc.get_sparse_core_info()
    nc, ns = info.num_cores, info.num_subcores
    nw = nc * ns
    n = ids_flat.shape[0]
    per_w = n // nw
    k = per_w // _CH
    steps = k // _G
    mesh = plsc.VectorSubcoreMesh(core_axis_name="c", subcore_axis_name="s")

    @functools.partial(
        pl.kernel, mesh=mesh,
        out_type=jax.ShapeDtypeStruct((n, _D), jnp.float32),
        scratch_types=[
            pltpu.VMEM((_G * _CH,), jnp.int32),
            pltpu.VMEM((_G, _CH, _D), jnp.float32),
            pltpu.SemaphoreType.DMA,
        ],
    )
    def gk(ids_hbm, table_hbm, out_hbm, idx_v, rows_v, sem):
        wid = lax.axis_index("s") * nc + lax.axis_index("c")
        tbase = wid * per_w

        def body(i, carry):
            pltpu.sync_copy(
                ids_hbm.at[pl.ds(tbase + i * (_G * _CH), _G * _CH)], idx_v)
            cps = [
                pltpu.async_copy(
                    table_hbm.at[idx_v.at[pl.ds(b * _CH, _CH)]],
                    rows_v.at[b], sem)
                for b in range(_G)
            ]
            for b in range(_G):
                cps[b].wait()
                pltpu.sync_copy(
                    rows_v.at[b],
                    out_hbm.at[pl.ds(tbase + (i * _G + b) * _CH, _CH)])
            return carry

        lax.fori_loop(0, steps, body, 0)

    return gk(ids_flat, table)


_RB = 8
_TW = 512


def _tc_body(x_ref, y_ref, a_ref, d_ref, k_ref, wsin_ref, wcos_ref, bs_ref,
             wall_ref, ball_ref, reg_ref, out_ref):
    colmask = lax.broadcasted_iota(jnp.int32, (1, _D), 1) == 0
    dn0 = (((0,), (0,)), ((), ()))
    for r in range(_RB):
        sl = pl.ds(r * _TW, _TW)
        xv = x_ref[r:r + 1, :].astype(jnp.bfloat16).astype(jnp.float32)
        yv = y_ref[r:r + 1, :].astype(jnp.bfloat16).astype(jnp.float32)
        z = k_ref[:, 0:1] * xv + k_ref[:, 1:2] * yv
        nrot = jnp.floor(z * 0.15915494309189535 + 0.5)
        z = (z - nrot * 6.28125) - nrot * 1.9353071795864769e-3
        loc = lax.dot_general(jnp.sin(z).astype(jnp.bfloat16), wsin_ref[...],
                              dn0, preferred_element_type=jnp.float32)
        loc += lax.dot_general(jnp.cos(z).astype(jnp.bfloat16), wcos_ref[...],
                               dn0, preferred_element_type=jnp.float32)
        out_ref[sl, 0:_D] = loc + bs_ref[...]
        aa = lax.dot_general(a_ref[r:r + 1, :], wall_ref[...], dn0,
                             precision=lax.Precision.HIGHEST,
                             preferred_element_type=jnp.float32) + ball_ref[...]
        out_ref[sl, _D:2 * _D] = jnp.where(colmask, aa, jnp.sin(aa))
        dd = lax.dot_general(d_ref[r:r + 1, :], wall_ref[...], dn0,
                             precision=lax.Precision.HIGHEST,
                             preferred_element_type=jnp.float32) + ball_ref[...]
        out_ref[sl, 2 * _D:3 * _D] = jnp.where(colmask, dd, jnp.sin(dd))
    out_ref[:, 3 * _D:4 * _D] = reg_ref[...]


def kernel(region_id, x, y, arrival_time, departure_time, region_table,
           Ws, bs, w0, b0, Wt, Bt):
    b_dim, s_dim = x.shape
    n = b_dim * s_dim

    reg = _sc_gather(region_table, region_id.reshape(n))

    scales = _LAMBDA_MIN * (_LAMBDA_MAX / _LAMBDA_MIN) ** (
        np.arange(_NUM_SCALES, dtype=np.float32) / max(_NUM_SCALES - 1, 1))
    angles = np.array([0.0, 2.0 * np.pi / 3.0, 4.0 * np.pi / 3.0],
                      dtype=np.float32)
    dirs = np.stack([np.cos(angles), np.sin(angles)], axis=-1)
    dirs_b = dirs.astype(ml_dtypes.bfloat16).astype(np.float32)
    kmat = (dirs_b[None, :, :].astype(np.float64)
            / scales[:, None, None].astype(np.float64)).reshape(
        3 * _NUM_SCALES, 2).astype(np.float32)
    kmat = jnp.asarray(kmat)

    ws3 = Ws.reshape(_NUM_SCALES, 6, _D)
    ws_sin = ws3[:, :3, :].reshape(3 * _NUM_SCALES, _D).astype(jnp.bfloat16)
    ws_cos = ws3[:, 3:, :].reshape(3 * _NUM_SCALES, _D).astype(jnp.bfloat16)
    w_all = jnp.concatenate([w0[None], Wt]).reshape(1, _D)
    b_all = jnp.concatenate([b0[None], Bt]).reshape(1, _D)

    t_blk = _RB * _TW
    grid = n // t_blk
    rows = lambda v: v.reshape(n // _TW, _TW)
    tok_spec = pl.BlockSpec((_RB, _TW), lambda i: (i, 0))
    full = lambda r, c: pl.BlockSpec((r, c), lambda i: (0, 0))

    out = pl.pallas_call(
        _tc_body,
        grid=(grid,),
        in_specs=[
            tok_spec, tok_spec, tok_spec, tok_spec,
            full(3 * _NUM_SCALES, 2),
            full(3 * _NUM_SCALES, _D),
            full(3 * _NUM_SCALES, _D),
            full(1, _D), full(1, _D), full(1, _D),
            pl.BlockSpec((t_blk, _D), lambda i: (i, 0)),
        ],
        out_specs=pl.BlockSpec((t_blk, 4 * _D), lambda i: (i, 0)),
        out_shape=jax.ShapeDtypeStruct((n, 4 * _D), jnp.float32),
    )(rows(x), rows(y), rows(arrival_time), rows(departure_time),
      kmat, ws_sin, ws_cos, bs.reshape(1, _D), w_all, b_all, reg)

    return out.reshape(b_dim, s_dim, 4 * _D)


# speedup vs baseline: 6.0349x; 2.0861x over previous
import functools

import jax
import jax.numpy as jnp
import ml_dtypes
import numpy as np
from jax import lax
from jax.experimental import pallas as pl
from jax.experimental.pallas import tpu as pltpu
from jax.experimental.pallas import tpu_sc as plsc

_D = 128
_NUM_SCALES = _D // 6
_LAMBDA_MIN = 0.01
_LAMBDA_MAX = 10.0
_CH = 128
_G = 5
_T = 2048


def _sc_gather(table, ids_flat):
    info = plsc.get_sparse_core_info()
    nc, ns = info.num_cores, info.num_subcores
    nw = nc * ns
    n = ids_flat.shape[0]
    per_w = n // nw
    k = per_w // _CH
    steps = k // _G
    mesh = plsc.VectorSubcoreMesh(core_axis_name="c", subcore_axis_name="s")

    @functools.partial(
        pl.kernel, mesh=mesh,
        out_type=jax.ShapeDtypeStruct((n, _D), jnp.float32),
        scratch_types=[
            pltpu.VMEM((_G * _CH,), jnp.int32),
            pltpu.VMEM((_G, _CH, _D), jnp.float32),
            pltpu.SemaphoreType.DMA,
        ],
    )
    def gk(ids_hbm, table_hbm, out_hbm, idx_v, rows_v, sem):
        wid = lax.axis_index("s") * nc + lax.axis_index("c")
        tbase = wid * per_w

        def body(i, carry):
            pltpu.sync_copy(
                ids_hbm.at[pl.ds(tbase + i * (_G * _CH), _G * _CH)], idx_v)
            cps = [
                pltpu.async_copy(
                    table_hbm.at[idx_v.at[pl.ds(b * _CH, _CH)]],
                    rows_v.at[b], sem)
                for b in range(_G)
            ]
            for b in range(_G):
                cps[b].wait()
                pltpu.sync_copy(
                    rows_v.at[b],
                    out_hbm.at[pl.ds(tbase + (i * _G + b) * _CH, _CH)])
            return carry

        lax.fori_loop(0, steps, body, 0)

    return gk(ids_flat, table)


_RB = 8
_TW = 512

_MAGIC = 12582912.0
_INV_HPI = 0.6366197723675814
_HPI_HI = 1.5703125
_HPI_LO = 4.8382679489659615e-4


def _sincos_core(z):
    t = z * _INV_HPI + _MAGIC
    q = lax.bitcast_convert_type(t, jnp.int32)
    kf = t - _MAGIC
    r = z - kf * _HPI_HI
    r = r - kf * _HPI_LO
    r2 = r * r
    s = r * (1.0 + r2 * (-1.6666667e-1 + r2 * (8.3333333e-3
                                               + r2 * (-1.9841270e-4))))
    c = 1.0 + r2 * (-0.5 + r2 * (4.1666667e-2 + r2 * (-1.3888889e-3
                                                      + r2 * 2.4801587e-5)))
    return s, c, q


def _flip_sign(v, sign_bits):
    return lax.bitcast_convert_type(
        lax.bitcast_convert_type(v, jnp.int32) ^ sign_bits, jnp.float32)


def _fast_sin(z):
    s, c, q = _sincos_core(z)
    return _flip_sign(jnp.where((q & 1) == 0, s, c), (q & 2) << 30)


def _fast_sincos(z):
    s, c, q = _sincos_core(z)
    even = (q & 1) == 0
    sin_z = _flip_sign(jnp.where(even, s, c), (q & 2) << 30)
    cos_z = _flip_sign(jnp.where(even, c, s), ((q + 1) & 2) << 30)
    return sin_z, cos_z


def _tc_body(x_ref, y_ref, a_ref, d_ref, k_ref, wsin_ref, wcos_ref, bs_ref,
             wall_ref, ball_ref, reg_ref, out_ref):
    colmask = lax.broadcasted_iota(jnp.int32, (1, _D), 1) == 0
    dn0 = (((0,), (0,)), ((), ()))
    for r in range(_RB):
        sl = pl.ds(r * _TW, _TW)
        xv = x_ref[r:r + 1, :].astype(jnp.bfloat16).astype(jnp.float32)
        yv = y_ref[r:r + 1, :].astype(jnp.bfloat16).astype(jnp.float32)
        z = k_ref[:, 0:1] * xv + k_ref[:, 1:2] * yv
        sin_z, cos_z = _fast_sincos(z)
        loc = lax.dot_general(sin_z.astype(jnp.bfloat16), wsin_ref[...],
                              dn0, preferred_element_type=jnp.float32)
        loc += lax.dot_general(cos_z.astype(jnp.bfloat16), wcos_ref[...],
                               dn0, preferred_element_type=jnp.float32)
        out_ref[sl, 0:_D] = loc + bs_ref[...]
        aa = lax.dot_general(a_ref[r:r + 1, :], wall_ref[...], dn0,
                             precision=lax.Precision.HIGHEST,
                             preferred_element_type=jnp.float32) + ball_ref[...]
        out_ref[sl, _D:2 * _D] = jnp.where(colmask, aa, _fast_sin(aa))
        dd = lax.dot_general(d_ref[r:r + 1, :], wall_ref[...], dn0,
                             precision=lax.Precision.HIGHEST,
                             preferred_element_type=jnp.float32) + ball_ref[...]
        out_ref[sl, 2 * _D:3 * _D] = jnp.where(colmask, dd, _fast_sin(dd))
    out_ref[:, 3 * _D:4 * _D] = reg_ref[...]


def kernel(region_id, x, y, arrival_time, departure_time, region_table,
           Ws, bs, w0, b0, Wt, Bt):
    b_dim, s_dim = x.shape
    n = b_dim * s_dim

    reg = _sc_gather(region_table, region_id.reshape(n))

    scales = _LAMBDA_MIN * (_LAMBDA_MAX / _LAMBDA_MIN) ** (
        np.arange(_NUM_SCALES, dtype=np.float32) / max(_NUM_SCALES - 1, 1))
    angles = np.array([0.0, 2.0 * np.pi / 3.0, 4.0 * np.pi / 3.0],
                      dtype=np.float32)
    dirs = np.stack([np.cos(angles), np.sin(angles)], axis=-1)
    dirs_b = dirs.astype(ml_dtypes.bfloat16).astype(np.float32)
    kmat = (dirs_b[None, :, :].astype(np.float64)
            / scales[:, None, None].astype(np.float64)).reshape(
        3 * _NUM_SCALES, 2).astype(np.float32)
    kmat = jnp.asarray(kmat)

    ws3 = Ws.reshape(_NUM_SCALES, 6, _D)
    ws_sin = ws3[:, :3, :].reshape(3 * _NUM_SCALES, _D).astype(jnp.bfloat16)
    ws_cos = ws3[:, 3:, :].reshape(3 * _NUM_SCALES, _D).astype(jnp.bfloat16)
    w_all = jnp.concatenate([w0[None], Wt]).reshape(1, _D)
    b_all = jnp.concatenate([b0[None], Bt]).reshape(1, _D)

    t_blk = _RB * _TW
    grid = n // t_blk
    rows = lambda v: v.reshape(n // _TW, _TW)
    tok_spec = pl.BlockSpec((_RB, _TW), lambda i: (i, 0))
    full = lambda r, c: pl.BlockSpec((r, c), lambda i: (0, 0))

    out = pl.pallas_call(
        _tc_body,
        grid=(grid,),
        in_specs=[
            tok_spec, tok_spec, tok_spec, tok_spec,
            full(3 * _NUM_SCALES, 2),
            full(3 * _NUM_SCALES, _D),
            full(3 * _NUM_SCALES, _D),
            full(1, _D), full(1, _D), full(1, _D),
            pl.BlockSpec((t_blk, _D), lambda i: (i, 0)),
        ],
        out_specs=pl.BlockSpec((t_blk, 4 * _D), lambda i: (i, 0)),
        out_shape=jax.ShapeDtypeStruct((n, 4 * _D), jnp.float32),
    )(rows(x), rows(y), rows(arrival_time), rows(departure_time),
      kmat, ws_sin, ws_cos, bs.reshape(1, _D), w_all, b_all, reg)

    return out.reshape(b_dim, s_dim, 4 * _D)
